# baseline (speedup 1.0000x reference)
import functools

import jax
import jax.numpy as jnp
from jax import lax
from jax.experimental import pallas as pl
from jax.experimental.pallas import tpu as pltpu
from jax.experimental.pallas import tpu_sc as plsc

N = 10000
E = 320000
F_IN = 128
F_EDGE = 16
H = 64
L = 4
G = 128

NC = 2
NS = 16
TILES = NC * NS
EPT = E // TILES
CH = 80
NCHUNK = EPT // CH
NP_ = 10240
RPT = NP_ // NS
ZR = 88
VLANES = 16
AW = 80
NPC = 5120
NPD = 5632
RPTA = NPD // NS
RPTO = NPC // NS


def _pack_bf16_pairs(umn):
    a = jax.lax.bitcast_convert_type(umn[:, :32], jnp.int32)
    b = jax.lax.bitcast_convert_type(umn[:, 32:], jnp.int32)
    ar = (a + 0x7FFF + ((a >> 16) & 1)) >> 16
    br = b + 0x7FFF + ((b >> 16) & 1)
    return (ar & 0xFFFF) | (br & jnp.int32(-65536))


def _head_body(x_ref, batch_ref, initW_ref, initb_ref, W1_ref, b1_ref,
               W2_ref, b2_ref, Wm0_ref, bm0_ref,
               out0_ref, u0_ref, um0_ref, pnt_ref):
    iota_g = lax.broadcasted_iota(jnp.int32, (G, N), 0)
    oh = (batch_ref[...] == iota_g).astype(jnp.float32)
    cnt = jnp.sum(oh, axis=1, keepdims=True)
    pnt = oh / jnp.maximum(cnt, 1.0)
    pnt_ref[...] = pnt
    x = x_ref[...]
    g0 = jnp.dot(pnt, x, preferred_element_type=jnp.float32)
    o = jnp.dot(g0, W1_ref[...], preferred_element_type=jnp.float32) + b1_ref[...]
    o = o + jax.nn.relu(
        jnp.dot(o, W2_ref[...], preferred_element_type=jnp.float32) + b2_ref[...])
    out0_ref[...] = o
    u0 = jnp.dot(x, initW_ref[...], preferred_element_type=jnp.float32) + initb_ref[...]
    u0_ref[...] = u0
    um0 = jnp.dot(u0, Wm0_ref[...], preferred_element_type=jnp.float32) + bm0_ref[...]
    um0_ref[...] = jnp.concatenate([um0, um0], axis=1)


def _head_call(x, batch2, initW, initb, W1, b1, W2, b2, Wm0, bm0):
    return pl.pallas_call(
        _head_body,
        out_shape=[
            jax.ShapeDtypeStruct((G, H), jnp.float32),
            jax.ShapeDtypeStruct((N, H), jnp.float32),
            jax.ShapeDtypeStruct((N, 2 * H), jnp.float32),
            jax.ShapeDtypeStruct((G, N), jnp.float32),
        ],
    )(x, batch2, initW, initb, W1, b1, W2, b2, Wm0, bm0)


EB2 = 3200
NBLK = E // (2 * EB2)


def _edge_body(eaA_ref, eaB_ref, We_ref, be_ref, e2_ref):
    ea = jnp.concatenate([eaA_ref[...], eaB_ref[...]], axis=1)
    e = lax.dot_general(ea, We_ref[...], (((0,), (0,)), ((), ())),
                        preferred_element_type=jnp.float32) + be_ref[...]
    e2_ref[...] = jnp.concatenate([e[:EB2], e[EB2:]], axis=1)


def _edge_call(eaT, We, be):
    return pl.pallas_call(
        _edge_body,
        grid=(NBLK,),
        in_specs=[
            pl.BlockSpec((F_EDGE, EB2), lambda j: (0, 2 * j)),
            pl.BlockSpec((F_EDGE, EB2), lambda j: (0, 2 * j + 1)),
            pl.BlockSpec((F_EDGE, H), lambda j: (0, 0)),
            pl.BlockSpec((1, H), lambda j: (0, 0)),
        ],
        out_specs=pl.BlockSpec((EB2, 2 * H), lambda j: (j, 0)),
        out_shape=jax.ShapeDtypeStruct((E // 2, 2 * H), jnp.float32),
    )(eaT, eaT, We, be)


UMS0 = 632

def _sc_body(um2, e2, src3, dst3, agg_out,
             srcall, dstall, gbuf0, gbuf1, ebuf0, ebuf1, mbuf, zbuf,
             agg_sh, gsem0, gsem1, esem0, esem1):
    c = lax.axis_index("c")
    s = lax.axis_index("s")
    wid = c * NS + s
    ebase2 = wid * (EPT // 2)
    lo = c * NPC

    zero16 = jnp.zeros((VLANES,), jnp.float32)
    one16 = jnp.ones((VLANES,), jnp.float32)
    dummy16 = NPC + lax.iota(jnp.int32, VLANES) * 8

    pltpu.sync_copy(src3.at[wid], srcall)
    pltpu.sync_copy(dst3.at[wid], dstall)

    def drow(i, _):
        for j in range(CH // VLANES):
            sl = pl.ds(VLANES * j, VLANES)
            lv = dstall[i, sl] - lo
            ok = (lv >= 0) & (lv < NPC)
            dstall[i, sl] = jnp.where(ok, lv, dummy16)
        return 0

    lax.fori_loop(0, NCHUNK, drow, 0)

    def zrow(i, _):
        for j in range(AW // VLANES):
            zbuf[i, pl.ds(VLANES * j, VLANES)] = zero16
        return 0

    lax.fori_loop(0, ZR, zrow, 0)

    def orow(i, _):
        mbuf[i, pl.ds(H, VLANES)] = one16
        return 0

    lax.fori_loop(0, CH, orow, 0)

    for k in range(RPTA // ZR):
        row0 = s * RPTA + k * ZR
        pltpu.sync_copy(zbuf, agg_sh.at[pl.ds(row0, ZR), :])

    plsc.subcore_barrier()

    def issue(ci, gb, eb, gsem, esem):
        pltpu.async_copy(um2.at[srcall.at[ci]], gb, gsem)
        base2 = pl.multiple_of(ebase2 + ci * (CH // 2), 8)
        pltpu.async_copy(e2.at[pl.ds(base2, CH // 2), :], eb, esem)

    def wait_ge(gb, eb, gsem, esem):
        pltpu.make_async_copy(um2.at[pl.ds(0, CH), :], gb, gsem).wait()
        pltpu.make_async_copy(e2.at[pl.ds(0, CH // 2), :], eb, esem).wait()

    def compute_scatter(ci, gb, eb):
        def mrow(k2, _):
            for r in range(4):
                i2 = 4 * k2 + r
                for pe in range(2):
                    edge = 2 * i2 + pe
                    off = H * pe
                    for q in range(4):
                        fsl = pl.ds(VLANES * q, VLANES)
                        mbuf[edge, fsl] = (eb[i2, pl.ds(off + VLANES * q, VLANES)]
                                           * gb[edge, fsl])
            return 0

        lax.fori_loop(0, CH // 8, mrow, 0)
        pltpu.sync_copy(mbuf, agg_sh.at[dstall.at[ci]], add=True)

    def step(ci, cur, nxt, issue_next):
        (gb, eb, gsem, esem) = cur
        wait_ge(gb, eb, gsem, esem)
        if issue_next:
            issue(ci + 1, *nxt)
        compute_scatter(ci, gb, eb)

    B0 = (gbuf0, ebuf0, gsem0, esem0)
    B1 = (gbuf1, ebuf1, gsem1, esem1)

    issue(0, *B0)

    def pair(k, _):
        step(2 * k, B0, B1, True)
        step(2 * k + 1, B1, B0, True)
        return 0

    lax.fori_loop(0, (NCHUNK - 1) // 2, pair, 0)
    step(NCHUNK - 1, B0, B1, False)

    plsc.subcore_barrier()

    row0 = pl.multiple_of(s * RPTO, 8)
    orow0 = pl.multiple_of(c * NPC + s * RPTO, 8)
    pltpu.sync_copy(agg_sh.at[pl.ds(row0, RPTO), :],
                    agg_out.at[pl.ds(orow0, RPTO), :])


def _sc_call(um2, e2, src3, dst3):
    mesh = plsc.VectorSubcoreMesh(core_axis_name="c", subcore_axis_name="s")
    fn = pl.kernel(
        _sc_body,
        mesh=mesh,
        out_type=jax.ShapeDtypeStruct((NP_, AW), jnp.float32),
        scratch_types=[
            pltpu.VMEM((NCHUNK, CH), jnp.int32),
            pltpu.VMEM((NCHUNK, CH), jnp.int32),
            pltpu.VMEM((CH, 2 * H), jnp.float32),
            pltpu.VMEM((CH, 2 * H), jnp.float32),
            pltpu.VMEM((CH // 2, 2 * H), jnp.float32),
            pltpu.VMEM((CH // 2, 2 * H), jnp.float32),
            pltpu.VMEM((CH, AW), jnp.float32),
            pltpu.VMEM((ZR, AW), jnp.float32),
            pltpu.VMEM_SHARED((NPD, AW), jnp.float32),
            pltpu.SemaphoreType.DMA,
            pltpu.SemaphoreType.DMA,
            pltpu.SemaphoreType.DMA,
            pltpu.SemaphoreType.DMA,
        ],
    )
    return fn(um2, e2, src3, dst3)


def _combine_body(agg2_ref, u_ref, pnt_ref, outp_ref,
                  Wu_ref, Wa_ref, bu_ref, W1_ref, b1_ref, W2_ref, b2_ref,
                  g_ref, bt_ref, Wmn_ref, bmn_ref,
                  ub_ref, umn_ref, outn_ref):
    agg_full = agg2_ref[:N]
    deg = agg_full[:, H:H + 1]
    rdeg = 1.0 / jnp.maximum(deg, 1.0)
    agg = agg_full[:, :H] * rdeg
    u = u_ref[...]
    unew = jax.nn.relu(
        jnp.dot(u, Wu_ref[...], preferred_element_type=jnp.float32)
        + jnp.dot(agg, Wa_ref[...], preferred_element_type=jnp.float32)
        + bu_ref[...]) + u
    gpool = jnp.dot(pnt_ref[...], unew, preferred_element_type=jnp.float32)
    o = jnp.dot(gpool, W1_ref[...], preferred_element_type=jnp.float32) + b1_ref[...]
    o = o + jax.nn.relu(
        jnp.dot(o, W2_ref[...], preferred_element_type=jnp.float32) + b2_ref[...])
    outn_ref[...] = outp_ref[...] + o * (1.0 / L)

    mean = jnp.mean(unew, axis=0, keepdims=True)
    var = jnp.mean((unew - mean) ** 2, axis=0, keepdims=True)
    ub = (unew - mean) / jnp.sqrt(var + 1e-5) * g_ref[...] + bt_ref[...]
    ub_ref[...] = ub
    umn = jnp.dot(ub, Wmn_ref[...], preferred_element_type=jnp.float32) + bmn_ref[...]
    umn_ref[...] = jnp.concatenate([umn, umn], axis=1)


def _combine_call(agg2, u, pnt, outp, Wu, Wa, bu, fe, gn, btn, Wmn, bmn):
    return pl.pallas_call(
        _combine_body,
        out_shape=[
            jax.ShapeDtypeStruct((N, H), jnp.float32),
            jax.ShapeDtypeStruct((N, 2 * H), jnp.float32),
            jax.ShapeDtypeStruct((G, H), jnp.float32),
        ],
    )(agg2, u, pnt, outp, Wu, Wa, bu.reshape(1, H),
      fe["W1"], fe["b1"].reshape(1, H), fe["W2"], fe["b2"].reshape(1, H),
      gn.reshape(1, H), btn.reshape(1, H), Wmn, bmn.reshape(1, H))


def _final_body(outp_ref, aW_ref, ab_ref, fW_ref, fb_ref, res_ref):
    outn = outp_ref[...]
    h = jnp.dot(outn, aW_ref[...], preferred_element_type=jnp.float32) + ab_ref[...]
    res = jnp.dot(jax.nn.relu(h) + outn, fW_ref[...],
                  preferred_element_type=jnp.float32) + fb_ref[...]
    res_ref[...] = res


def _final_call(outp, p):
    return pl.pallas_call(
        _final_body,
        out_shape=jax.ShapeDtypeStruct((G, 1), jnp.float32),
    )(outp, p["after_W"], p["after_b"].reshape(1, H),
      p["final_W"], p["final_b"].reshape(1, 1))


def kernel(x, edge_index, edge_attr, batch, params):
    p = params
    src3 = (edge_index[0].reshape(NBLK, 2, EB2).transpose(0, 2, 1)
            .reshape(TILES, NCHUNK, CH))
    dst3 = (edge_index[1].reshape(NBLK, 2, EB2).transpose(0, 2, 1)
            .reshape(TILES, NCHUNK, CH))
    batch2 = batch.reshape(1, N)
    np_ = p["no_prop"]
    out0, u0, um0, pnt = _head_call(
        x, batch2, p["init_W"], p["init_b"].reshape(1, H),
        np_["W1"], np_["b1"].reshape(1, H), np_["W2"], np_["b2"].reshape(1, H),
        p["Wm"][0], p["bm"][0].reshape(1, H))

    eaT = edge_attr.T
    Wa_all = p["Wa"]
    gn_all = jnp.roll(p["bn_gamma"], -1, axis=0)
    btn_all = jnp.roll(p["bn_beta"], -1, axis=0)
    Wmn_all = jnp.roll(p["Wm"], -1, axis=0)
    bmn_all = jnp.roll(p["bm"], -1, axis=0)
    fe = p["fe"]

    def body(carry, xs):
        u, ump, out = carry
        Wei, bei, Wui, Wai, bui, gni, btni, Wmni, bmni = xs
        e2 = _edge_call(eaT, Wei, bei)
        agg2 = _sc_call(ump, e2, src3, dst3)
        u2, ump2, out2 = _combine_call(agg2, u, pnt, out, Wui, Wai, bui, fe,
                                       gni, btni, Wmni, bmni)
        return (u2, ump2, out2), None

    (_, _, outf), _ = lax.scan(
        body, (u0, um0, out0),
        (p["We"], p["be"].reshape(L, 1, H), p["Wu"], Wa_all, p["bu"],
         gn_all, btn_all, Wmn_all, bmn_all))

    res = _final_call(outf, p)
    return res[:, 0]


# speedup vs baseline: 1.1829x; 1.1829x over previous
import functools

import jax
import jax.numpy as jnp
from jax import lax
from jax.experimental import pallas as pl
from jax.experimental.pallas import tpu as pltpu
from jax.experimental.pallas import tpu_sc as plsc

N = 10000
E = 320000
F_IN = 128
F_EDGE = 16
H = 64
L = 4
G = 128

NC = 2
NS = 16
TILES = NC * NS
EPT = E // TILES
CH = 80
NCHUNK = EPT // CH
NP_ = 10240
RPT = NP_ // NS
ZR = 88
VLANES = 16
AW = 80
NPC = 5120
NPD = 5632
RPTA = NPD // NS
RPTO = NPC // NS


def _pack_bf16_pairs(umn):
    a = jax.lax.bitcast_convert_type(umn[:, :32], jnp.int32)
    b = jax.lax.bitcast_convert_type(umn[:, 32:], jnp.int32)
    ar = (a + 0x7FFF + ((a >> 16) & 1)) >> 16
    br = b + 0x7FFF + ((b >> 16) & 1)
    return (ar & 0xFFFF) | (br & jnp.int32(-65536))


def _head_body(x_ref, batch_ref, initW_ref, initb_ref, W1_ref, b1_ref,
               W2_ref, b2_ref, Wm0_ref, bm0_ref,
               out0_ref, u0_ref, um0_ref, pnt_ref):
    iota_g = lax.broadcasted_iota(jnp.int32, (G, N), 0)
    oh = (batch_ref[...] == iota_g).astype(jnp.float32)
    cnt = jnp.sum(oh, axis=1, keepdims=True)
    pnt = oh / jnp.maximum(cnt, 1.0)
    pnt_ref[...] = pnt
    x = x_ref[...]
    g0 = jnp.dot(pnt, x, preferred_element_type=jnp.float32)
    o = jnp.dot(g0, W1_ref[...], preferred_element_type=jnp.float32) + b1_ref[...]
    o = o + jax.nn.relu(
        jnp.dot(o, W2_ref[...], preferred_element_type=jnp.float32) + b2_ref[...])
    out0_ref[...] = o
    u0 = jnp.dot(x, initW_ref[...], preferred_element_type=jnp.float32) + initb_ref[...]
    u0_ref[...] = u0
    um0 = jnp.dot(u0, Wm0_ref[...], preferred_element_type=jnp.float32) + bm0_ref[...]
    um0_ref[...] = jnp.concatenate([um0, um0], axis=1)


def _head_call(x, batch2, initW, initb, W1, b1, W2, b2, Wm0, bm0):
    return pl.pallas_call(
        _head_body,
        out_shape=[
            jax.ShapeDtypeStruct((G, H), jnp.float32),
            jax.ShapeDtypeStruct((N, H), jnp.float32),
            jax.ShapeDtypeStruct((N, 2 * H), jnp.float32),
            jax.ShapeDtypeStruct((G, N), jnp.float32),
        ],
    )(x, batch2, initW, initb, W1, b1, W2, b2, Wm0, bm0)


EB2 = 3200
NBLK = E // (2 * EB2)


def _edge_body(eaA_ref, eaB_ref, We_ref, be_ref, e2_ref):
    ea = jnp.concatenate([eaA_ref[...], eaB_ref[...]], axis=1)
    e = lax.dot_general(ea, We_ref[...], (((0,), (0,)), ((), ())),
                        preferred_element_type=jnp.float32) + be_ref[...]
    e2_ref[...] = jnp.concatenate([e[:EB2], e[EB2:]], axis=1)


def _edge_call(eaT, We, be):
    return pl.pallas_call(
        _edge_body,
        grid=(NBLK,),
        in_specs=[
            pl.BlockSpec((F_EDGE, EB2), lambda j: (0, 2 * j)),
            pl.BlockSpec((F_EDGE, EB2), lambda j: (0, 2 * j + 1)),
            pl.BlockSpec((F_EDGE, H), lambda j: (0, 0)),
            pl.BlockSpec((1, H), lambda j: (0, 0)),
        ],
        out_specs=pl.BlockSpec((EB2, 2 * H), lambda j: (j, 0)),
        out_shape=jax.ShapeDtypeStruct((E // 2, 2 * H), jnp.float32),
    )(eaT, eaT, We, be)


UMS0 = 632

def _sc_body(um2, e2, src3, dst3, agg_out,
             srcall, dstall, gbuf0, gbuf1, ebuf0, ebuf1, mbuf, zbuf,
             agg_sh, gsem0, gsem1, esem0, esem1):
    c = lax.axis_index("c")
    s = lax.axis_index("s")
    wid = c * NS + s
    ebase2 = wid * (EPT // 2)
    lo = c * NPC

    zero16 = jnp.zeros((VLANES,), jnp.float32)
    one16 = jnp.ones((VLANES,), jnp.float32)
    dummy16 = NPC + lax.iota(jnp.int32, VLANES) * 8

    pltpu.sync_copy(src3.at[wid], srcall)
    pltpu.sync_copy(dst3.at[wid], dstall)

    def drow(i, _):
        for j in range(CH // VLANES):
            sl = pl.ds(VLANES * j, VLANES)
            lv = dstall[i, sl] - lo
            ok = (lv >= 0) & (lv < NPC)
            dstall[i, sl] = jnp.where(ok, lv, dummy16)
        return 0

    lax.fori_loop(0, NCHUNK, drow, 0)

    def zrow(i, _):
        for j in range(AW // VLANES):
            zbuf[i, pl.ds(VLANES * j, VLANES)] = zero16
        return 0

    lax.fori_loop(0, ZR, zrow, 0)

    def orow(i, _):
        mbuf[i, pl.ds(H, VLANES)] = one16
        return 0

    lax.fori_loop(0, CH, orow, 0)

    for k in range(RPTA // ZR):
        row0 = s * RPTA + k * ZR
        pltpu.sync_copy(zbuf, agg_sh.at[pl.ds(row0, ZR), :])

    plsc.subcore_barrier()

    def issue(ci, gb, eb, gsem, esem):
        pltpu.async_copy(um2.at[srcall.at[ci]], gb, gsem)
        base2 = pl.multiple_of(ebase2 + ci * (CH // 2), 8)
        pltpu.async_copy(e2.at[pl.ds(base2, CH // 2), :], eb, esem)

    def wait_ge(gb, eb, gsem, esem):
        pltpu.make_async_copy(um2.at[pl.ds(0, CH), :], gb, gsem).wait()
        pltpu.make_async_copy(e2.at[pl.ds(0, CH // 2), :], eb, esem).wait()

    def compute_scatter(ci, gb, eb):
        def mrow(k2, _):
            for r in range(4):
                i2 = 4 * k2 + r
                for pe in range(2):
                    edge = 2 * i2 + pe
                    off = H * pe
                    for q in range(4):
                        fsl = pl.ds(VLANES * q, VLANES)
                        mbuf[edge, fsl] = (eb[i2, pl.ds(off + VLANES * q, VLANES)]
                                           * gb[edge, fsl])
            return 0

        lax.fori_loop(0, CH // 8, mrow, 0)
        pltpu.sync_copy(mbuf, agg_sh.at[dstall.at[ci]], add=True)

    def step(ci, cur, nxt, issue_next):
        (gb, eb, gsem, esem) = cur
        wait_ge(gb, eb, gsem, esem)
        if issue_next:
            issue(ci + 1, *nxt)
        compute_scatter(ci, gb, eb)

    B0 = (gbuf0, ebuf0, gsem0, esem0)
    B1 = (gbuf1, ebuf1, gsem1, esem1)

    issue(0, *B0)

    def pair(k, _):
        step(2 * k, B0, B1, True)
        step(2 * k + 1, B1, B0, True)
        return 0

    lax.fori_loop(0, (NCHUNK - 1) // 2, pair, 0)
    step(NCHUNK - 1, B0, B1, False)

    plsc.subcore_barrier()

    row0 = pl.multiple_of(s * RPTO, 8)
    orow0 = pl.multiple_of(c * NPC + s * RPTO, 8)
    pltpu.sync_copy(agg_sh.at[pl.ds(row0, RPTO), :],
                    agg_out.at[pl.ds(orow0, RPTO), :])


def _sc_call(um2, e2, src3, dst3):
    mesh = plsc.VectorSubcoreMesh(core_axis_name="c", subcore_axis_name="s")
    fn = pl.kernel(
        _sc_body,
        mesh=mesh,
        out_type=jax.ShapeDtypeStruct((NP_, AW), jnp.float32),
        scratch_types=[
            pltpu.VMEM((NCHUNK, CH), jnp.int32),
            pltpu.VMEM((NCHUNK, CH), jnp.int32),
            pltpu.VMEM((CH, 2 * H), jnp.float32),
            pltpu.VMEM((CH, 2 * H), jnp.float32),
            pltpu.VMEM((CH // 2, 2 * H), jnp.float32),
            pltpu.VMEM((CH // 2, 2 * H), jnp.float32),
            pltpu.VMEM((CH, AW), jnp.float32),
            pltpu.VMEM((ZR, AW), jnp.float32),
            pltpu.VMEM_SHARED((NPD, AW), jnp.float32),
            pltpu.SemaphoreType.DMA,
            pltpu.SemaphoreType.DMA,
            pltpu.SemaphoreType.DMA,
            pltpu.SemaphoreType.DMA,
        ],
    )
    return fn(um2, e2, src3, dst3)


def _combine_body(agg2_ref, u_ref, pnt_ref, outp_ref,
                  Wu_ref, Wa_ref, bu_ref, W1_ref, b1_ref, W2_ref, b2_ref,
                  g_ref, bt_ref, Wmn_ref, bmn_ref,
                  ub_ref, umn_ref, outn_ref):
    agg_full = agg2_ref[:N]
    deg = agg_full[:, H:H + 1]
    rdeg = 1.0 / jnp.maximum(deg, 1.0)
    agg = agg_full[:, :H] * rdeg
    u = u_ref[...]
    unew = jax.nn.relu(
        jnp.dot(u, Wu_ref[...], preferred_element_type=jnp.float32)
        + jnp.dot(agg, Wa_ref[...], preferred_element_type=jnp.float32)
        + bu_ref[...]) + u
    gpool = jnp.dot(pnt_ref[...], unew, preferred_element_type=jnp.float32)
    o = jnp.dot(gpool, W1_ref[...], preferred_element_type=jnp.float32) + b1_ref[...]
    o = o + jax.nn.relu(
        jnp.dot(o, W2_ref[...], preferred_element_type=jnp.float32) + b2_ref[...])
    outn_ref[...] = outp_ref[...] + o * (1.0 / L)

    mean = jnp.mean(unew, axis=0, keepdims=True)
    var = jnp.mean((unew - mean) ** 2, axis=0, keepdims=True)
    ub = (unew - mean) / jnp.sqrt(var + 1e-5) * g_ref[...] + bt_ref[...]
    ub_ref[...] = ub
    umn = jnp.dot(ub, Wmn_ref[...], preferred_element_type=jnp.float32) + bmn_ref[...]
    umn_ref[...] = jnp.concatenate([umn, umn], axis=1)


def _combine_call(agg2, u, pnt, outp, Wu, Wa, bu, fe, gn, btn, Wmn, bmn):
    return pl.pallas_call(
        _combine_body,
        out_shape=[
            jax.ShapeDtypeStruct((N, H), jnp.float32),
            jax.ShapeDtypeStruct((N, 2 * H), jnp.float32),
            jax.ShapeDtypeStruct((G, H), jnp.float32),
        ],
    )(agg2, u, pnt, outp, Wu, Wa, bu.reshape(1, H),
      fe["W1"], fe["b1"].reshape(1, H), fe["W2"], fe["b2"].reshape(1, H),
      gn.reshape(1, H), btn.reshape(1, H), Wmn, bmn.reshape(1, H))


def _final_body(outp_ref, aW_ref, ab_ref, fW_ref, fb_ref, res_ref):
    outn = outp_ref[...]
    h = jnp.dot(outn, aW_ref[...], preferred_element_type=jnp.float32) + ab_ref[...]
    res = jnp.dot(jax.nn.relu(h) + outn, fW_ref[...],
                  preferred_element_type=jnp.float32) + fb_ref[...]
    res_ref[...] = res


def _final_call(outp, p):
    return pl.pallas_call(
        _final_body,
        out_shape=jax.ShapeDtypeStruct((G, 1), jnp.float32),
    )(outp, p["after_W"], p["after_b"].reshape(1, H),
      p["final_W"], p["final_b"].reshape(1, 1))


def kernel(x, edge_index, edge_attr, batch, params):
    p = params
    src3 = (edge_index[0].reshape(NBLK, 2, EB2).transpose(0, 2, 1)
            .reshape(TILES, NCHUNK, CH))
    dst3 = (edge_index[1].reshape(NBLK, 2, EB2).transpose(0, 2, 1)
            .reshape(TILES, NCHUNK, CH))
    batch2 = batch.reshape(1, N)
    np_ = p["no_prop"]
    out0, u0, um0, pnt = _head_call(
        x, batch2, p["init_W"], p["init_b"].reshape(1, H),
        np_["W1"], np_["b1"].reshape(1, H), np_["W2"], np_["b2"].reshape(1, H),
        p["Wm"][0], p["bm"][0].reshape(1, H))

    eaT = edge_attr.T
    Wa_all = p["Wa"]
    fe = p["fe"]

    u, um, out = u0, um0, out0
    for i in range(L):
        e2 = _edge_call(eaT, p["We"][i], p["be"][i].reshape(1, H))
        agg2 = _sc_call(um, e2, src3, dst3)
        j = (i + 1) % L
        u, um, out = _combine_call(agg2, u, pnt, out, p["Wu"][i], Wa_all[i],
                                   p["bu"][i], fe, p["bn_gamma"][j],
                                   p["bn_beta"][j], p["Wm"][j], p["bm"][j])

    res = _final_call(out, p)
    return res[:, 0]
